# baseline (speedup 1.0000x reference)
import functools

import jax
import jax.numpy as jnp
from jax import lax
from jax.experimental import pallas as pl
from jax.experimental.pallas import tpu as pltpu
from jax.experimental.pallas import tpu_sc as plsc

VOCAB = 65024
DIM = 1024
BATCH = 4
SEQ = 8192

_INFO = plsc.get_sparse_core_info()
_NC = _INFO.num_cores
_NS = _INFO.num_subcores
_NW = _NC * _NS
_B = BATCH * SEQ
_BPW = _B // _NW
_WPB = SEQ // _BPW
_C = 16
_NCH = _BPW // _C
_ROUNDS = _NCH // 2


def _body(idx_hbm, table_hbm, out_hbm, idx_v, buf0, buf1, gs0, gs1, ws0, ws1):
    wid = lax.axis_index("s") * _NC + lax.axis_index("c")
    row = wid // _WPB
    col = (wid % _WPB) * _BPW
    bufs = (buf0, buf1)
    gsems = (gs0, gs1)
    wsems = (ws0, ws1)

    pltpu.sync_copy(idx_hbm.at[row, pl.ds(col, _BPW)], idx_v)

    def gather_start(ci, b):
        pltpu.async_copy(
            table_hbm.at[idx_v.at[pl.ds(ci * _C, _C)]], bufs[b], gsems[b])

    def gather_wait(ci, b):
        pltpu.make_async_copy(
            table_hbm.at[idx_v.at[pl.ds(ci * _C, _C)]], bufs[b], gsems[b]).wait()

    def write_start(ci, b):
        pltpu.async_copy(
            bufs[b], out_hbm.at[row, pl.ds(col + ci * _C, _C)], wsems[b])

    def write_wait(ci, b):
        pltpu.make_async_copy(
            bufs[b], out_hbm.at[row, pl.ds(col + ci * _C, _C)], wsems[b]).wait()

    gather_start(0, 0)
    gather_start(1, 1)

    def round_(g, carry):
        i0 = g * 2
        gather_wait(i0, 0)
        write_start(i0, 0)
        gather_wait(i0 + 1, 1)
        write_start(i0 + 1, 1)
        write_wait(i0, 0)
        gather_start(i0 + 2, 0)
        write_wait(i0 + 1, 1)
        gather_start(i0 + 3, 1)
        return carry

    lax.fori_loop(0, _ROUNDS - 1, round_, 0)
    last = _NCH - 2
    gather_wait(last, 0)
    write_start(last, 0)
    gather_wait(last + 1, 1)
    write_start(last + 1, 1)
    write_wait(last, 0)
    write_wait(last + 1, 1)


@jax.jit
def _embed(ids, table):
    run = functools.partial(
        pl.kernel,
        out_type=jax.ShapeDtypeStruct((BATCH, SEQ, DIM), jnp.float32),
        mesh=plsc.VectorSubcoreMesh(core_axis_name="c", subcore_axis_name="s"),
        scratch_types=[
            pltpu.VMEM((_BPW,), jnp.int32),
            pltpu.VMEM((_C, DIM), jnp.float32),
            pltpu.VMEM((_C, DIM), jnp.float32),
            pltpu.SemaphoreType.DMA,
            pltpu.SemaphoreType.DMA,
            pltpu.SemaphoreType.DMA,
            pltpu.SemaphoreType.DMA,
        ],
    )(_body)
    return run(ids, table)


def kernel(input_ids, embed_table):
    if input_ids.dtype != jnp.int32:
        input_ids = input_ids.astype(jnp.int32)
    return _embed(input_ids, embed_table)


# speedup vs baseline: 1.0440x; 1.0440x over previous
import functools

import jax
import jax.numpy as jnp
from jax import lax
from jax.experimental import pallas as pl
from jax.experimental.pallas import tpu as pltpu
from jax.experimental.pallas import tpu_sc as plsc

VOCAB = 65024
DIM = 1024
BATCH = 4
SEQ = 8192

_INFO = plsc.get_sparse_core_info()
_NC = _INFO.num_cores
_NS = _INFO.num_subcores
_NW = _NC * _NS
_B = BATCH * SEQ
_BPW = _B // _NW
_WPB = SEQ // _BPW
_C = 32
_NCH = _BPW // _C
_ROUNDS = _NCH // 2


def _body(idx_hbm, table_hbm, out_hbm, idx_v, buf0, buf1, gs0, gs1, ws0, ws1):
    wid = lax.axis_index("s") * _NC + lax.axis_index("c")
    row = wid // _WPB
    col = (wid % _WPB) * _BPW
    bufs = (buf0, buf1)
    gsems = (gs0, gs1)
    wsems = (ws0, ws1)

    pltpu.sync_copy(idx_hbm.at[row, pl.ds(col, _BPW)], idx_v)

    def gather_start(ci, b):
        pltpu.async_copy(
            table_hbm.at[idx_v.at[pl.ds(ci * _C, _C)]], bufs[b], gsems[b])

    def gather_wait(ci, b):
        pltpu.make_async_copy(
            table_hbm.at[idx_v.at[pl.ds(ci * _C, _C)]], bufs[b], gsems[b]).wait()

    def write_start(ci, b):
        pltpu.async_copy(
            bufs[b], out_hbm.at[row, pl.ds(col + ci * _C, _C)], wsems[b])

    def write_wait(ci, b):
        pltpu.make_async_copy(
            bufs[b], out_hbm.at[row, pl.ds(col + ci * _C, _C)], wsems[b]).wait()

    gather_start(0, 0)
    gather_start(1, 1)

    def round_(g, carry):
        i0 = g * 2
        gather_wait(i0, 0)
        write_start(i0, 0)
        gather_wait(i0 + 1, 1)
        write_start(i0 + 1, 1)
        write_wait(i0, 0)
        gather_start(i0 + 2, 0)
        write_wait(i0 + 1, 1)
        gather_start(i0 + 3, 1)
        return carry

    lax.fori_loop(0, _ROUNDS - 1, round_, 0)
    last = _NCH - 2
    gather_wait(last, 0)
    write_start(last, 0)
    gather_wait(last + 1, 1)
    write_start(last + 1, 1)
    write_wait(last, 0)
    write_wait(last + 1, 1)


@jax.jit
def _embed(ids, table):
    run = functools.partial(
        pl.kernel,
        out_type=jax.ShapeDtypeStruct((BATCH, SEQ, DIM), jnp.float32),
        mesh=plsc.VectorSubcoreMesh(core_axis_name="c", subcore_axis_name="s"),
        scratch_types=[
            pltpu.VMEM((_BPW,), jnp.int32),
            pltpu.VMEM((_C, DIM), jnp.float32),
            pltpu.VMEM((_C, DIM), jnp.float32),
            pltpu.SemaphoreType.DMA,
            pltpu.SemaphoreType.DMA,
            pltpu.SemaphoreType.DMA,
            pltpu.SemaphoreType.DMA,
        ],
    )(_body)
    return run(ids, table)


def kernel(input_ids, embed_table):
    if input_ids.dtype != jnp.int32:
        input_ids = input_ids.astype(jnp.int32)
    return _embed(input_ids, embed_table)


# speedup vs baseline: 1.0879x; 1.0420x over previous
import functools

import jax
import jax.numpy as jnp
from jax import lax
from jax.experimental import pallas as pl
from jax.experimental.pallas import tpu as pltpu
from jax.experimental.pallas import tpu_sc as plsc

VOCAB = 65024
DIM = 1024
BATCH = 4
SEQ = 8192

_INFO = plsc.get_sparse_core_info()
_NC = _INFO.num_cores
_NS = _INFO.num_subcores
_NW = _NC * _NS
_B = BATCH * SEQ
_BPW = _B // _NW
_WPB = SEQ // _BPW
_C = 32
_NCH = _BPW // _C
_ROUNDS = _NCH // 2


def _body(idx_hbm, table_hbm, out_hbm, idx_v,
          buf0, buf1, buf2, gs0, gs1, gs2, ws0, ws1, ws2):
    wid = lax.axis_index("s") * _NC + lax.axis_index("c")
    row = wid // _WPB
    col = (wid % _WPB) * _BPW
    bufs = (buf0, buf1, buf2)
    gsems = (gs0, gs1, gs2)
    wsems = (ws0, ws1, ws2)

    pltpu.sync_copy(idx_hbm.at[row, pl.ds(col, _BPW)], idx_v)

    def gather_start(ci, b):
        pltpu.async_copy(
            table_hbm.at[idx_v.at[pl.ds(ci * _C, _C)]], bufs[b], gsems[b])

    def gather_wait(ci, b):
        pltpu.make_async_copy(
            table_hbm.at[idx_v.at[pl.ds(ci * _C, _C)]], bufs[b], gsems[b]).wait()

    def write_start(ci, b):
        pltpu.async_copy(
            bufs[b], out_hbm.at[row, pl.ds(col + ci * _C, _C)], wsems[b])

    def write_wait(ci, b):
        pltpu.make_async_copy(
            bufs[b], out_hbm.at[row, pl.ds(col + ci * _C, _C)], wsems[b]).wait()

    gather_start(0, 0)
    gather_start(1, 1)
    gather_start(2, 2)

    def round_(g, carry):
        c = g * 3
        gather_wait(c, 0)
        write_start(c, 0)
        gather_wait(c + 1, 1)
        write_start(c + 1, 1)
        write_wait(c, 0)
        gather_start(c + 3, 0)
        gather_wait(c + 2, 2)
        write_start(c + 2, 2)
        write_wait(c + 1, 1)
        gather_start(c + 4, 1)
        write_wait(c + 2, 2)
        gather_start(c + 5, 2)
        return carry

    lax.fori_loop(0, (_NCH - 5) // 3, round_, 0)
    c = _NCH - 5
    gather_wait(c, 0)
    write_start(c, 0)
    gather_wait(c + 1, 1)
    write_start(c + 1, 1)
    write_wait(c, 0)
    gather_start(c + 3, 0)
    gather_wait(c + 2, 2)
    write_start(c + 2, 2)
    write_wait(c + 1, 1)
    gather_start(c + 4, 1)
    gather_wait(c + 3, 0)
    write_start(c + 3, 0)
    gather_wait(c + 4, 1)
    write_start(c + 4, 1)
    write_wait(c + 2, 2)
    write_wait(c + 3, 0)
    write_wait(c + 4, 1)


@jax.jit
def _embed(ids, table):
    run = functools.partial(
        pl.kernel,
        out_type=jax.ShapeDtypeStruct((BATCH, SEQ, DIM), jnp.float32),
        mesh=plsc.VectorSubcoreMesh(core_axis_name="c", subcore_axis_name="s"),
        scratch_types=(
            [pltpu.VMEM((_BPW,), jnp.int32)]
            + [pltpu.VMEM((_C, DIM), jnp.float32)] * 3
            + [pltpu.SemaphoreType.DMA] * 6
        ),
    )(_body)
    return run(ids, table)


def kernel(input_ids, embed_table):
    if input_ids.dtype != jnp.int32:
        input_ids = input_ids.astype(jnp.int32)
    return _embed(input_ids, embed_table)
